# speedup vs baseline: 30.5633x; 30.5633x over previous
import functools

import jax
import jax.numpy as jnp
from jax import lax
from jax.experimental import pallas as pl
from jax.experimental.pallas import tpu as pltpu

HEADS = 8
DIM_HEAD = 64
N_POINTS = 8
DIM = 768
INNER = HEADS * DIM_HEAD
GRID = 32
HW = GRID * GRID


def _gelu_exact(x):
    return 0.5 * x * (1.0 + lax.erf(x * (2.0 ** -0.5)))


def _body(x_ref, xoff_ref, ctx_ref, Wv_ref, oW1_ref, ob1_ref, oW2_ref,
          ob2_ref, aW1_ref, ab1_ref, aW2_ref, ab2_ref, pW_ref, pb_ref,
          out_ref):
    xb = x_ref[0]
    xo = xoff_ref[0]
    ctx = ctx_ref[0]

    v = jnp.dot(ctx, Wv_ref[...], preferred_element_type=jnp.float32)

    h_aw = _gelu_exact(jnp.dot(xb, aW1_ref[...],
                               preferred_element_type=jnp.float32) + ab1_ref[...])
    logits = jnp.dot(h_aw, aW2_ref[...],
                     preferred_element_type=jnp.float32) + ab2_ref[...]
    e = jnp.exp(logits)
    ci = lax.broadcasted_iota(jnp.int32, (64, 64), 0) // N_POINTS
    cj = lax.broadcasted_iota(jnp.int32, (64, 64), 1) // N_POINTS
    S = (ci == cj).astype(jnp.float32)
    denom = jnp.dot(e, S, preferred_element_type=jnp.float32)
    attw = e / denom

    h_off = _gelu_exact(jnp.dot(xo, oW1_ref[...],
                                preferred_element_type=jnp.float32) + ob1_ref[...])
    off = jnp.tanh(jnp.dot(h_off, oW2_ref[...],
                           preferred_element_type=jnp.float32) + ob2_ref[...])

    gx_grid = (lax.broadcasted_iota(jnp.int32, (1, HW), 1) %
               GRID).astype(jnp.float32)
    gy_grid = (lax.broadcasted_iota(jnp.int32, (1, HW), 1) //
               GRID).astype(jnp.float32)

    half = (GRID - 1) * 0.5
    outs = []
    for h in range(HEADS):
        acc = None
        for p in range(N_POINTS):
            c = h * (N_POINTS * 2) + p * 2
            gx = off[:, c:c + 1]
            gy = off[:, c + 1:c + 2]
            ix = (gx + 1.0) * half
            iy = (gy + 1.0) * half
            a = attw[:, h * N_POINTS + p:h * N_POINTS + p + 1]
            hx = jnp.maximum(0.0, 1.0 - jnp.abs(gx_grid - ix))
            hy = jnp.maximum(0.0, 1.0 - jnp.abs(gy_grid - iy))
            t = (a * hx) * hy
            acc = t if acc is None else acc + t
        vh = v[:, h * DIM_HEAD:(h + 1) * DIM_HEAD]
        outs.append(jnp.dot(acc, vh, preferred_element_type=jnp.float32))
    sampled = jnp.concatenate(outs, axis=1)

    out_ref[0] = (jnp.dot(sampled, pW_ref[...],
                          preferred_element_type=jnp.float32) + pb_ref[...])


@jax.jit
def kernel(x, context, W_q, W_v, off_W1, off_b1, off_W2, off_b2,
           aw_W1, aw_b1, aw_W2, aw_b2, out_W, out_b):
    B, N, _ = x.shape
    x_perm = jnp.transpose(x.reshape(4, 16, 4, DIM), (2, 0, 1, 3)).reshape(
        B, N, DIM)

    full = lambda *s: pl.BlockSpec(s, lambda b: (0,) * len(s))
    out = pl.pallas_call(
        _body,
        grid=(B,),
        in_specs=[
            pl.BlockSpec((1, N, DIM), lambda b: (b, 0, 0)),
            pl.BlockSpec((1, N, DIM), lambda b: (b, 0, 0)),
            pl.BlockSpec((1, HW, DIM), lambda b: (b, 0, 0)),
            full(DIM, INNER),
            full(DIM, DIM),
            full(1, DIM),
            full(DIM, HEADS * N_POINTS * 2),
            full(1, HEADS * N_POINTS * 2),
            full(DIM, DIM),
            full(1, DIM),
            full(DIM, HEADS * N_POINTS),
            full(1, HEADS * N_POINTS),
            full(INNER, DIM),
            full(1, DIM),
        ],
        out_specs=pl.BlockSpec((1, N, DIM), lambda b: (b, 0, 0)),
        out_shape=jax.ShapeDtypeStruct((B, N, DIM), jnp.float32),
    )(x, x_perm, context, W_v,
      off_W1, off_b1.reshape(1, -1), off_W2, off_b2.reshape(1, -1),
      aw_W1, aw_b1.reshape(1, -1), aw_W2, aw_b2.reshape(1, -1),
      out_W, out_b.reshape(1, -1))
    return out
